# baseline (speedup 1.0000x reference)
import math

import jax
import jax.numpy as jnp
import numpy as np
from jax.experimental import pallas as pl
from jax.experimental.pallas import tpu as pltpu

_SIZE = 100000
_CONF = 0.9
_EPS = float(np.float32(0.1 / (_SIZE - 2)))
_H = _CONF * math.log(_CONF) + (_SIZE - 2) * _EPS * math.log(_EPS)
_R = 16
_NBUF = 4


def _body(t_hbm, x_hbm, o_ref, t_v, bufs, sems, tsem):
    n_blocks = 1024 // _R

    pltpu.make_async_copy(t_hbm, t_v, tsem).start()

    def start(i, b):
        pltpu.make_async_copy(
            x_hbm.at[pl.ds(i * _R, _R), :], bufs.at[b], sems.at[b]
        ).start()

    def wait(b):
        pltpu.make_async_copy(
            x_hbm.at[pl.ds(0, _R), :], bufs.at[b], sems.at[b]
        ).wait()

    for b in range(_NBUF):
        start(b, b)

    pltpu.make_async_copy(t_hbm, t_v, tsem).wait()

    def step(i, acc):
        b = jax.lax.rem(i, _NBUF)
        wait(b)
        x = bufs[b]
        t = t_v[pl.ds(i * _R, _R), :]
        w = (t != 0).astype(jnp.float32)
        iota = jax.lax.broadcasted_iota(jnp.int32, x.shape, 1)
        rowsum = jnp.sum(x, axis=1, keepdims=True)
        vk = jnp.sum(jnp.where(iota == t, x, 0.0), axis=1, keepdims=True)
        contrib = jnp.sum(
            w * (_H - (_CONF - _EPS) * vk - _EPS * (rowsum - x[:, 0:1]))
        )

        nxt = i + _NBUF

        @pl.when(nxt < n_blocks)
        def _():
            start(nxt, b)

        return acc + contrib

    o_ref[0, 0] = jax.lax.fori_loop(0, n_blocks, step, 0.0)


def kernel(x, target):
    n = x.shape[0]
    t2d = target.astype(jnp.int32).reshape(n, 1)
    out = pl.pallas_call(
        _body,
        in_specs=[
            pl.BlockSpec(memory_space=pl.ANY),
            pl.BlockSpec(memory_space=pl.ANY),
        ],
        out_specs=pl.BlockSpec(memory_space=pltpu.SMEM),
        out_shape=jax.ShapeDtypeStruct((1, 1), jnp.float32),
        scratch_shapes=[
            pltpu.VMEM((n, 1), jnp.int32),
            pltpu.VMEM((_NBUF, _R, _SIZE), jnp.float32),
            pltpu.SemaphoreType.DMA((_NBUF,)),
            pltpu.SemaphoreType.DMA,
        ],
    )(t2d, x)
    return out[0, 0]
